# baseline (speedup 1.0000x reference)
import functools

import jax
import jax.numpy as jnp
from jax import lax
from jax.experimental import pallas as pl
from jax.experimental.pallas import tpu as pltpu
from jax.experimental.pallas import tpu_sc as plsc

_NC = 2
_NS = 16
_NW = _NC * _NS
_L = 16


def _project_body(field_ref, matrix_ref, t_ref):
    m = matrix_ref[...]
    mp = jnp.concatenate(
        [m[:, 0:1], m[:, 2:3], m[:, 3:4], m[:, 1:2]], axis=1
    )
    t_ref[...] = lax.dot_general(
        mp, field_ref[...], (((1,), (1,)), ((), ())),
        preferred_element_type=jnp.float32,
    )


@functools.cache
def _make_project(g, p):
    return pl.pallas_call(
        _project_body,
        out_shape=jax.ShapeDtypeStruct((p, g), jnp.float32),
    )


_TILE = 128


@functools.cache
def _make_gather(n, g, p):
    assert n % _L == 0 and p % 8 == 0
    nt = -(-n // _TILE)
    tpw = -(-nt // _NW)
    bpw = tpw * _TILE
    tail_t = nt - (_NW - 1) * tpw
    assert 0 < tail_t <= tpw
    tail_rows = n - (_NW - 1) * bpw
    assert 0 < tail_rows <= tail_t * _TILE and tail_rows % _L == 0
    mesh = plsc.VectorSubcoreMesh(
        core_axis_name="c", subcore_axis_name="s",
        num_cores=_NC, num_subcores=_NS,
    )

    @functools.partial(
        pl.kernel,
        out_type=jax.ShapeDtypeStruct((p // 8, nt, 8, _TILE), jnp.float32),
        mesh=mesh,
        scratch_types=[
            pltpu.VMEM((p * g,), jnp.float32),
            pltpu.VMEM((bpw,), jnp.int32),
            pltpu.VMEM((p, bpw), jnp.float32),
            pltpu.SemaphoreType.DMA,
            pltpu.SemaphoreType.DMA,
        ],
        compiler_params=pltpu.CompilerParams(
            use_tc_tiling_on_sc=False, needs_layout_passes=False,
        ),
    )
    def gather(t_hbm, idx_hbm, out_hbm, t_v, idx_v, blk_v, sem, sem2):
        wid = lax.axis_index("s") * _NC + lax.axis_index("c")
        base = wid * bpw
        tbase = wid * tpw
        is_tail = wid == _NW - 1
        tiles_w = jnp.where(is_tail, tail_t, tpw)

        c_t = pltpu.async_copy(t_hbm, t_v, sem2)

        @pl.when(jnp.logical_not(is_tail))
        def _stage_main():
            pltpu.async_copy(idx_hbm.at[pl.ds(base, bpw)],
                             idx_v.at[pl.ds(0, bpw)], sem).wait()

        @pl.when(is_tail)
        def _stage_tail():
            pltpu.async_copy(idx_hbm.at[pl.ds(base, tail_rows)],
                             idx_v.at[pl.ds(0, tail_rows)], sem).wait()
            zeros = jnp.zeros((_L,), jnp.int32)
            for q in range(tail_rows, tail_t * _TILE, _L):
                idx_v[pl.ds(q, _L)] = zeros

        c_t.wait()

        def tile(j, carry):
            coff = j * _TILE
            for q in range(_TILE // _L):
                off = coff + q * _L
                idx = idx_v[pl.ds(off, _L)]
                for pp in range(p):
                    vals = plsc.load_gather(t_v, [idx + pp * g])
                    blk_v[pp, pl.ds(off, _L)] = vals
            for pt in range(p // 8):
                pltpu.async_copy(
                    blk_v.at[pl.ds(pt * 8, 8), pl.ds(coff, _TILE)],
                    out_hbm.at[pt, tbase + j],
                    sem,
                )
            return carry

        lax.fori_loop(0, tiles_w, tile, 0)

        def drain(i, carry):
            pltpu.make_async_copy(
                out_hbm.at[0, 0],
                blk_v.at[pl.ds(0, 8), pl.ds(0, _TILE)],
                sem,
            ).wait()
            return carry

        lax.fori_loop(0, tiles_w * (p // 8), drain, 0)

    return gather


def kernel(batch, positions, field, matrix):
    del positions
    n = batch.shape[0]
    g = field.shape[0]
    p = matrix.shape[0]

    t = _make_project(g, p)(field, matrix)
    idx = batch.astype(jnp.int32)
    out4 = _make_gather(n, g, p)(t.reshape(-1), idx)
    nt = out4.shape[1]
    return out4.transpose(1, 3, 0, 2).reshape(nt * _TILE, p)[:n]


# speedup vs baseline: 1.2613x; 1.2613x over previous
import functools

import jax
import jax.numpy as jnp
from jax import lax
from jax.experimental import pallas as pl
from jax.experimental.pallas import tpu as pltpu
from jax.experimental.pallas import tpu_sc as plsc

_NC = 2
_NS = 16
_NW = _NC * _NS
_L = 16


def _project_body(field_ref, matrix_ref, t_ref):
    m = matrix_ref[...]
    mp = jnp.concatenate(
        [m[:, 0:1], m[:, 2:3], m[:, 3:4], m[:, 1:2]], axis=1
    )
    t_ref[...] = lax.dot_general(
        mp, field_ref[...], (((1,), (1,)), ((), ())),
        preferred_element_type=jnp.float32,
    )


@functools.cache
def _make_project(g, p):
    return pl.pallas_call(
        _project_body,
        out_shape=jax.ShapeDtypeStruct((p, g), jnp.float32),
    )


_TILE = 128


@functools.cache
def _make_gather(n, g, p):
    assert n % _L == 0 and p % 8 == 0
    nt = -(-n // _TILE)
    tpw = -(-nt // _NW)
    bpw = tpw * _TILE
    tail_t = nt - (_NW - 1) * tpw
    assert 0 < tail_t <= tpw
    tail_rows = n - (_NW - 1) * bpw
    assert 0 < tail_rows <= tail_t * _TILE and tail_rows % _L == 0
    mesh = plsc.VectorSubcoreMesh(
        core_axis_name="c", subcore_axis_name="s",
        num_cores=_NC, num_subcores=_NS,
    )

    @functools.partial(
        pl.kernel,
        out_type=jax.ShapeDtypeStruct((p // 8, nt, 8, _TILE), jnp.float32),
        mesh=mesh,
        scratch_types=[
            pltpu.VMEM((p * g,), jnp.float32),
            pltpu.VMEM((bpw,), jnp.int32),
            pltpu.VMEM((p, bpw), jnp.float32),
            pltpu.SemaphoreType.DMA,
            pltpu.SemaphoreType.DMA,
        ],
        compiler_params=pltpu.CompilerParams(
            use_tc_tiling_on_sc=False, needs_layout_passes=False,
        ),
    )
    def gather(t_hbm, idx_hbm, out_hbm, t_v, idx_v, blk_v, sem, sem2):
        wid = lax.axis_index("s") * _NC + lax.axis_index("c")
        base = wid * bpw
        tbase = wid * tpw
        is_tail = wid == _NW - 1
        tiles_w = jnp.where(is_tail, tail_t, tpw)

        c_t = pltpu.async_copy(t_hbm, t_v, sem2)

        @pl.when(jnp.logical_not(is_tail))
        def _stage_main():
            pltpu.async_copy(idx_hbm.at[pl.ds(base, bpw)],
                             idx_v.at[pl.ds(0, bpw)], sem).wait()

        @pl.when(is_tail)
        def _stage_tail():
            pltpu.async_copy(idx_hbm.at[pl.ds(base, tail_rows)],
                             idx_v.at[pl.ds(0, tail_rows)], sem).wait()
            zeros = jnp.zeros((_L,), jnp.int32)

            def zfill(i, carry):
                idx_v[pl.ds(i * _L, _L)] = zeros
                return carry

            lax.fori_loop(tail_rows // _L, bpw // _L, zfill, 0)

        c_t.wait()

        def fire(j, carry):
            for pt in range(p // 8):
                pltpu.async_copy(
                    blk_v.at[pl.ds(pt * 8, 8), pl.ds(j * _TILE, _TILE)],
                    out_hbm.at[pt, tbase + j],
                    sem,
                )
            return carry

        groups = bpw // _L
        t_half = (groups // 2) * _L // _TILE
        for h in range(2):
            @functools.partial(
                plsc.parallel_loop, h * (groups // 2),
                groups if h else groups // 2, unroll=4,
            )
            def group(i):
                off = i * _L
                idx = idx_v[pl.ds(off, _L)]
                for pp in range(p):
                    vals = plsc.load_gather(t_v, [idx + pp * g])
                    blk_v[pp, pl.ds(off, _L)] = vals

            if h == 0:
                lax.fori_loop(0, jnp.minimum(tiles_w, t_half), fire, 0)
            else:
                lax.fori_loop(jnp.minimum(tiles_w, t_half), tiles_w, fire, 0)

        def drain(i, carry):
            pltpu.make_async_copy(
                out_hbm.at[0, 0],
                blk_v.at[pl.ds(0, 8), pl.ds(0, _TILE)],
                sem,
            ).wait()
            return carry

        lax.fori_loop(0, tiles_w * (p // 8), drain, 0)

    return gather


def kernel(batch, positions, field, matrix):
    del positions
    n = batch.shape[0]
    g = field.shape[0]
    p = matrix.shape[0]

    t = _make_project(g, p)(field, matrix)
    idx = batch.astype(jnp.int32)
    out4 = _make_gather(n, g, p)(t.reshape(-1), idx)
    nt = out4.shape[1]
    return out4.transpose(1, 3, 0, 2).reshape(nt * _TILE, p)[:n]
